# baseline (speedup 1.0000x reference)
import jax
import jax.numpy as jnp
from jax.experimental import pallas as pl
import jax.experimental.pallas.tpu as pltpu

B, S, H, E, F, TOPK = 32, 1, 2048, 8, 1024, 2
SWIGLU_SCALE = 1.702
T = B * S
FBLK = 512
NSTEP = E * (F // FBLK)
NBUF = 4


def _moe_kernel(x_ref, rw_ref, g_hbm, u_hbm, d_hbm, out_ref,
                gbuf, ubuf, dbuf, w_ref, gsem, usem, dsem):
    i = pl.program_id(0)

    def start(c, slot):
        e, h = c // 2, (c % 2) * FBLK
        pltpu.make_async_copy(g_hbm.at[e, :, pl.ds(h, FBLK)], gbuf.at[slot],
                              gsem.at[slot]).start()
        pltpu.make_async_copy(u_hbm.at[e, :, pl.ds(h, FBLK)], ubuf.at[slot],
                              usem.at[slot]).start()
        pltpu.make_async_copy(d_hbm.at[e, pl.ds(h, FBLK), :], dbuf.at[slot],
                              dsem.at[slot]).start()

    @pl.when(i == 0)
    def _prologue():
        for c in range(NBUF):
            start(c, c)
        x = x_ref[...]
        logits = jnp.dot(x, rw_ref[...], preferred_element_type=jnp.float32)
        m = jnp.max(logits, axis=-1, keepdims=True)
        p = jnp.exp(logits - m)
        aff = p / jnp.sum(p, axis=-1, keepdims=True)
        eids = jax.lax.broadcasted_iota(jnp.int32, (T, E), 1)
        i1 = jnp.argmax(aff, axis=-1, keepdims=True)
        v1 = jnp.max(aff, axis=-1, keepdims=True)
        masked = jnp.where(eids == i1, -jnp.inf, aff)
        i2 = jnp.argmax(masked, axis=-1, keepdims=True)
        v2 = jnp.max(masked, axis=-1, keepdims=True)
        s = v1 + v2
        w_ref[...] = jnp.where(eids == i1, v1 / s, 0.0) + jnp.where(
            eids == i2, v2 / s, 0.0)

    e = i // 2
    slot = jax.lax.rem(i, NBUF)
    x = x_ref[...]
    pltpu.make_async_copy(g_hbm.at[0, :, pl.ds(0, FBLK)], gbuf.at[slot],
                          gsem.at[slot]).wait()
    gate = jnp.dot(x, gbuf[slot], preferred_element_type=jnp.float32)
    pltpu.make_async_copy(u_hbm.at[0, :, pl.ds(0, FBLK)], ubuf.at[slot],
                          usem.at[slot]).wait()
    up = jnp.dot(x, ubuf[slot], preferred_element_type=jnp.float32)
    eids = jax.lax.broadcasted_iota(jnp.int32, (T, E), 1)
    w_col = jnp.sum(jnp.where(eids == e, w_ref[...], 0.0), axis=-1,
                    keepdims=True)
    act = (w_col * gate) * jax.nn.sigmoid(SWIGLU_SCALE * gate) * up
    pltpu.make_async_copy(d_hbm.at[0, pl.ds(0, FBLK), :], dbuf.at[slot],
                          dsem.at[slot]).wait()
    contrib = jnp.dot(act, dbuf[slot], preferred_element_type=jnp.float32)

    @pl.when(i == 0)
    def _init():
        out_ref[...] = contrib

    @pl.when(i != 0)
    def _acc():
        out_ref[...] += contrib

    @pl.when(i + NBUF < NSTEP)
    def _next():
        start(i + NBUF, slot)


def kernel(hidden_states, router_weight, gate_proj, up_proj, down_proj):
    x = hidden_states.reshape(T, H)
    out = pl.pallas_call(
        _moe_kernel,
        grid=(NSTEP,),
        in_specs=[
            pl.BlockSpec((T, H), lambda i: (0, 0)),
            pl.BlockSpec((H, E), lambda i: (0, 0)),
            pl.BlockSpec(memory_space=pltpu.MemorySpace.HBM),
            pl.BlockSpec(memory_space=pltpu.MemorySpace.HBM),
            pl.BlockSpec(memory_space=pltpu.MemorySpace.HBM),
        ],
        out_specs=pl.BlockSpec((T, H), lambda i: (0, 0)),
        out_shape=jax.ShapeDtypeStruct((T, H), jnp.float32),
        scratch_shapes=[
            pltpu.VMEM((NBUF, H, FBLK), jnp.float32),
            pltpu.VMEM((NBUF, H, FBLK), jnp.float32),
            pltpu.VMEM((NBUF, FBLK, H), jnp.float32),
            pltpu.VMEM((T, E), jnp.float32),
            pltpu.SemaphoreType.DMA((NBUF,)),
            pltpu.SemaphoreType.DMA((NBUF,)),
            pltpu.SemaphoreType.DMA((NBUF,)),
        ],
    )(x, router_weight, gate_proj, up_proj, down_proj)
    return out.reshape(B, S, H)


# speedup vs baseline: 1.0066x; 1.0066x over previous
import functools

import jax
import jax.numpy as jnp
from jax.experimental import pallas as pl
import jax.experimental.pallas.tpu as pltpu

B, S, H, E, F, TOPK = 32, 1, 2048, 8, 1024, 2
SWIGLU_SCALE = 1.702
FBLK = 512
NF = F // FBLK
T = B * S


def _moe_kernel(x_ref, rw_ref, g_ref, u_ref, d_ref, out_ref, w_ref):
    e = pl.program_id(0)
    f = pl.program_id(1)

    @pl.when((e == 0) & (f == 0))
    def _router():
        x = x_ref[...]
        logits = jnp.dot(x, rw_ref[...], preferred_element_type=jnp.float32)
        m = jnp.max(logits, axis=-1, keepdims=True)
        p = jnp.exp(logits - m)
        aff = p / jnp.sum(p, axis=-1, keepdims=True)
        eids = jax.lax.broadcasted_iota(jnp.int32, (T, E), 1)
        i1 = jnp.argmax(aff, axis=-1, keepdims=True)
        v1 = jnp.max(aff, axis=-1, keepdims=True)
        masked = jnp.where(eids == i1, -jnp.inf, aff)
        i2 = jnp.argmax(masked, axis=-1, keepdims=True)
        v2 = jnp.max(masked, axis=-1, keepdims=True)
        s = v1 + v2
        w_ref[...] = jnp.where(eids == i1, v1 / s, 0.0) + jnp.where(
            eids == i2, v2 / s, 0.0)
        out_ref[...] = jnp.zeros_like(out_ref)

    x = x_ref[...]
    gate = jnp.dot(x, g_ref[0], preferred_element_type=jnp.float32)
    up = jnp.dot(x, u_ref[0], preferred_element_type=jnp.float32)
    eids = jax.lax.broadcasted_iota(jnp.int32, (T, E), 1)
    w_col = jnp.sum(jnp.where(eids == e, w_ref[...], 0.0), axis=-1,
                    keepdims=True)
    act = (w_col * gate) * jax.nn.sigmoid(SWIGLU_SCALE * gate) * up
    contrib = jnp.dot(act, d_ref[0], preferred_element_type=jnp.float32)
    out_ref[...] += contrib


@functools.partial(jax.jit, static_argnames=())
def kernel(hidden_states, router_weight, gate_proj, up_proj, down_proj):
    x = hidden_states.reshape(T, H)
    out = pl.pallas_call(
        _moe_kernel,
        grid=(E, NF),
        in_specs=[
            pl.BlockSpec((T, H), lambda e, f: (0, 0)),
            pl.BlockSpec((H, E), lambda e, f: (0, 0)),
            pl.BlockSpec((1, H, FBLK), lambda e, f: (e, 0, f)),
            pl.BlockSpec((1, H, FBLK), lambda e, f: (e, 0, f)),
            pl.BlockSpec((1, FBLK, H), lambda e, f: (e, f, 0)),
        ],
        out_specs=pl.BlockSpec((T, H), lambda e, f: (0, 0)),
        out_shape=jax.ShapeDtypeStruct((T, H), jnp.float32),
        scratch_shapes=[pltpu.VMEM((T, E), jnp.float32)],
    )(x, router_weight, gate_proj, up_proj, down_proj)
    return out.reshape(B, S, H)


# speedup vs baseline: 1.0092x; 1.0025x over previous
import functools

import jax
import jax.numpy as jnp
from jax.experimental import pallas as pl
import jax.experimental.pallas.tpu as pltpu

B, S, H, E, F, TOPK = 32, 1, 2048, 8, 1024, 2
SWIGLU_SCALE = 1.702
FBLK = 512
NF = F // FBLK
T = B * S


def _moe_kernel(x_ref, rw_ref, g_ref, u_ref, d_ref, out_ref, w_ref):
    e = pl.program_id(0)
    f = pl.program_id(1)

    @pl.when((e == 0) & (f == 0))
    def _router():
        x = x_ref[...]
        logits = jnp.dot(x, rw_ref[...], preferred_element_type=jnp.float32)
        m = jnp.max(logits, axis=-1, keepdims=True)
        p = jnp.exp(logits - m)
        aff = p / jnp.sum(p, axis=-1, keepdims=True)
        eids = jax.lax.broadcasted_iota(jnp.int32, (T, E), 1)
        i1 = jnp.argmax(aff, axis=-1, keepdims=True)
        v1 = jnp.max(aff, axis=-1, keepdims=True)
        masked = jnp.where(eids == i1, -jnp.inf, aff)
        i2 = jnp.argmax(masked, axis=-1, keepdims=True)
        v2 = jnp.max(masked, axis=-1, keepdims=True)
        s = v1 + v2
        w_ref[...] = jnp.where(eids == i1, v1 / s, 0.0) + jnp.where(
            eids == i2, v2 / s, 0.0)
        out_ref[...] = jnp.zeros_like(out_ref)

    x = x_ref[...]
    gate = jnp.dot(x, g_ref[0], preferred_element_type=jnp.float32)
    up = jnp.dot(x, u_ref[0], preferred_element_type=jnp.float32)
    act = gate * jax.nn.sigmoid(SWIGLU_SCALE * gate) * up
    contrib = jnp.dot(act, d_ref[0], preferred_element_type=jnp.float32)
    eids = jax.lax.broadcasted_iota(jnp.int32, (T, E), 1)
    w_col = jnp.sum(jnp.where(eids == e, w_ref[...], 0.0), axis=-1,
                    keepdims=True)
    out_ref[...] += w_col * contrib


@functools.partial(jax.jit, static_argnames=())
def kernel(hidden_states, router_weight, gate_proj, up_proj, down_proj):
    x = hidden_states.reshape(T, H)
    out = pl.pallas_call(
        _moe_kernel,
        grid=(E, NF),
        in_specs=[
            pl.BlockSpec((T, H), lambda e, f: (0, 0)),
            pl.BlockSpec((H, E), lambda e, f: (0, 0)),
            pl.BlockSpec((1, H, FBLK), lambda e, f: (e, 0, f)),
            pl.BlockSpec((1, H, FBLK), lambda e, f: (e, 0, f)),
            pl.BlockSpec((1, FBLK, H), lambda e, f: (e, f, 0)),
        ],
        out_specs=pl.BlockSpec((T, H), lambda e, f: (0, 0)),
        out_shape=jax.ShapeDtypeStruct((T, H), jnp.float32),
        scratch_shapes=[pltpu.VMEM((T, E), jnp.float32)],
    )(x, router_weight, gate_proj, up_proj, down_proj)
    return out.reshape(B, S, H)
